# baseline (speedup 1.0000x reference)
import functools

import jax
import jax.numpy as jnp
from jax import lax
from jax.experimental import pallas as pl
from jax.experimental.pallas import tpu as pltpu
from jax.experimental.pallas import tpu_sc as plsc

N = 10000
DEG = 32
F = 128
TDIM = 16

NW = 32
NODES_PER_W = 320
NPAD = NW * NODES_PER_W
CH = 4
ROWS = CH * DEG
CHUNKS = NODES_PER_W // CH
NBUF = 4
IDX_PER_W = NODES_PER_W * DEG


def _sc_gather_sum(x, nbr_flat):
    info = plsc.get_sparse_core_info()
    nc = info.num_cores

    mesh = plsc.VectorSubcoreMesh(core_axis_name="c", subcore_axis_name="s")

    @functools.partial(
        pl.kernel,
        mesh=mesh,
        out_type=jax.ShapeDtypeStruct((NPAD, F), jnp.float32),
        scratch_types=[
            pltpu.VMEM((CHUNKS, ROWS), jnp.int32),
        ] + [pltpu.VMEM((ROWS, F), jnp.float32) for _ in range(NBUF)]
          + [pltpu.VMEM((CH, F), jnp.float32) for _ in range(NBUF)]
          + [pltpu.SemaphoreType.DMA for _ in range(2 * NBUF)],
    )
    def body(x_hbm, nbr_hbm, out_hbm, idx_v, *bufs):
        rows = bufs[:NBUF]
        pools = bufs[NBUF:2 * NBUF]
        sems = bufs[2 * NBUF:3 * NBUF]
        osems = bufs[3 * NBUF:4 * NBUF]
        wid = lax.axis_index("s") * nc + lax.axis_index("c")

        pltpu.sync_copy(nbr_hbm.at[pl.ds(wid * CHUNKS, CHUNKS)], idx_v)

        def desc(chunk_i, slot):
            ci = lax.min(chunk_i, CHUNKS - 1)
            return pltpu.make_async_copy(
                x_hbm.at[idx_v.at[ci]], rows[slot], sems[slot])

        def odesc(chunk_i, slot):
            base = wid * NODES_PER_W + lax.min(chunk_i, CHUNKS - 1) * CH
            return pltpu.make_async_copy(
                pools[slot], out_hbm.at[pl.ds(base, CH)], osems[slot])

        for b in range(NBUF):
            desc(b, b).start()

        def group(ig, carry):
            for b in range(NBUF):
                i = ig * NBUF + b
                desc(i, b).wait()
                rb = rows[b]
                pb = pools[b]
                @pl.when(ig > 0)
                def _():
                    odesc(i - NBUF, b).wait()
                def group(gi, carry):
                    c = gi // (F // 16)
                    col = gi % (F // 16)
                    sl = pl.ds(col * 16, 16)
                    base = c * DEG
                    a0 = rb[base + 0, sl]
                    a1 = rb[base + 1, sl]
                    for d in range(2, DEG, 2):
                        a0 = a0 + rb[base + d, sl]
                        a1 = a1 + rb[base + d + 1, sl]
                    pb[c, sl] = a0 + a1
                    return carry

                lax.fori_loop(0, CH * (F // 16), group, 0)
                odesc(i, b).start()
                desc(i + NBUF, b).start()
            return carry

        lax.fori_loop(0, CHUNKS // NBUF, group, 0)
        for b in range(NBUF):
            desc(CHUNKS + b, b).wait()
            odesc(CHUNKS - NBUF + b, b).wait()

    return body(x, nbr_flat)


BLK = 512
_PREC = lax.Precision.HIGHEST


def _tc_body(scal_ref, tt_ref, x_ref, g_ref, Wv_ref, Tv_ref, W1a_ref,
             W1b_ref, W2_ref, b1_ref, b2_ref, o_ref):
    tt = tt_ref[...]
    w0 = scal_ref[0, 0]
    b0 = scal_ref[0, 1]
    zrows = [jnp.sum(tt, axis=0, keepdims=True) * w0 + (DEG * b0)]
    for j in range(TDIM - 1):
        wt = scal_ref[0, 2 + j]
        bt = scal_ref[0, 2 + (TDIM - 1) + j]
        zrows.append(jnp.sum(jnp.sin(tt * wt + bt), axis=0, keepdims=True))
    Z = jnp.concatenate(zrows, axis=0)
    ht = lax.dot_general(Z, Tv_ref[...], (((0,), (0,)), ((), ())),
                         precision=_PREC, preferred_element_type=jnp.float32)
    h = jnp.dot(g_ref[...], Wv_ref[...], precision=_PREC,
                preferred_element_type=jnp.float32) + ht
    pre = (jnp.dot(x_ref[...], W1a_ref[...], precision=_PREC,
                   preferred_element_type=jnp.float32)
           + jnp.dot(h, W1b_ref[...], precision=_PREC,
                     preferred_element_type=jnp.float32)
           + b1_ref[...])
    o_ref[...] = (jnp.dot(jnp.maximum(pre, 0.0), W2_ref[...], precision=_PREC,
                          preferred_element_type=jnp.float32) + b2_ref[...])


def _tc_post(scal, times_t, x, g, Wv, Tv, W1a, W1b, W2, b1eff, b2r):
    grid = (pl.cdiv(N, BLK),)
    full = lambda shape: pl.BlockSpec(shape, lambda i: (0, 0))
    return pl.pallas_call(
        _tc_body,
        grid=grid,
        in_specs=[
            pl.BlockSpec(memory_space=pltpu.SMEM),
            pl.BlockSpec((DEG, BLK), lambda i: (0, i)),
            pl.BlockSpec((BLK, F), lambda i: (i, 0)),
            pl.BlockSpec((BLK, F), lambda i: (i, 0)),
            full((F, F)),
            full((TDIM, F)),
            full((F, F)),
            full((F, F)),
            full((F, F)),
            full((1, F)),
            full((1, F)),
        ],
        out_specs=pl.BlockSpec((BLK, F), lambda i: (i, 0)),
        out_shape=jax.ShapeDtypeStruct((N, F), jnp.float32),
    )(scal, times_t, x, g, Wv, Tv, W1a, W1b, W2, b1eff, b2r)


def kernel(x, neighbors, times, t, Wk, bk, Wq, bq, Wv, bv, w0, b0, Wt, Bt,
           Tk, btk, Tq, btq, Tv, btv, W1, b1, W2, b2):
    nbr = neighbors.astype(jnp.int32).reshape(-1)
    nbr_flat = jnp.pad(nbr, (0, (NPAD - N) * DEG)).reshape(NPAD // CH, ROWS)
    g = _sc_gather_sum(x, nbr_flat)[:N]

    times_t = jnp.pad(times.T, ((0, 0), (0, BLK * pl.cdiv(N, BLK) - N)))
    scal = jnp.concatenate(
        [w0.reshape(1), b0.reshape(1), Wt.reshape(TDIM - 1), Bt.reshape(TDIM - 1)]
    ).reshape(1, 2 * TDIM)
    W1a = W1[:F]
    W1b = W1[F:]
    b1eff = (b1 + (DEG * (bv + btv)) @ W1b).reshape(1, F)
    return _tc_post(scal, times_t, x, g, Wv, Tv, W1a, W1b, W2, b1eff,
                    b2.reshape(1, F))


# speedup vs baseline: 1.0868x; 1.0868x over previous
import functools

import jax
import jax.numpy as jnp
from jax import lax
from jax.experimental import pallas as pl
from jax.experimental.pallas import tpu as pltpu
from jax.experimental.pallas import tpu_sc as plsc

N = 10000
DEG = 32
F = 128
TDIM = 16

NW = 32
NODES_PER_W = 320
NPAD = NW * NODES_PER_W
CH = 4
ROWS = CH * DEG
CHUNKS = NODES_PER_W // CH
NBUF = 4
IDX_PER_W = NODES_PER_W * DEG
TCHUNKS = NPAD // CH
Q0 = 104
Q1 = 56


def _sc_gather_sum(x, nbr_flat):
    info = plsc.get_sparse_core_info()
    nc = info.num_cores

    mesh = plsc.VectorSubcoreMesh(core_axis_name="c", subcore_axis_name="s")

    @functools.partial(
        pl.kernel,
        mesh=mesh,
        out_type=jax.ShapeDtypeStruct((NPAD, F), jnp.float32),
        scratch_types=[
            pltpu.VMEM((Q0, ROWS), jnp.int32),
        ] + [pltpu.VMEM((ROWS, F), jnp.float32) for _ in range(NBUF)]
          + [pltpu.VMEM((CH, F), jnp.float32) for _ in range(NBUF)]
          + [pltpu.SemaphoreType.DMA for _ in range(2 * NBUF)],
    )
    def body(x_hbm, nbr_hbm, out_hbm, idx_v, *bufs):
        rows = bufs[:NBUF]
        pools = bufs[NBUF:2 * NBUF]
        sems = bufs[2 * NBUF:3 * NBUF]
        osems = bufs[3 * NBUF:4 * NBUF]
        sid = lax.axis_index("s")
        cid = lax.axis_index("c")
        q = lax.select(cid == 0, jnp.int32(Q0), jnp.int32(Q1))
        start_chunk = lax.select(cid == 0, sid * Q0, 16 * Q0 + sid * Q1)

        pltpu.sync_copy(nbr_hbm.at[pl.ds(start_chunk, Q0)], idx_v)

        def desc(chunk_i, slot):
            ci = lax.min(chunk_i, q - 1)
            return pltpu.make_async_copy(
                x_hbm.at[idx_v.at[ci]], rows[slot], sems[slot])

        def odesc(chunk_i, slot):
            base = (start_chunk + lax.min(chunk_i, q - 1)) * CH
            return pltpu.make_async_copy(
                pools[slot], out_hbm.at[pl.ds(base, CH)], osems[slot])

        for b in range(NBUF):
            desc(b, b).start()

        def group(ig, carry):
            for b in range(NBUF):
                i = ig * NBUF + b
                desc(i, b).wait()
                rb = rows[b]
                pb = pools[b]
                @pl.when(ig > 0)
                def _():
                    odesc(i - NBUF, b).wait()
                def red(gi, carry2):
                    c = gi // (F // 16)
                    col = gi % (F // 16)
                    sl = pl.ds(col * 16, 16)
                    base = c * DEG
                    a0 = rb[base + 0, sl]
                    a1 = rb[base + 1, sl]
                    for d in range(2, DEG, 2):
                        a0 = a0 + rb[base + d, sl]
                        a1 = a1 + rb[base + d + 1, sl]
                    pb[c, sl] = a0 + a1
                    return carry2

                lax.fori_loop(0, CH * (F // 16), red, 0)
                odesc(i, b).start()
                desc(i + NBUF, b).start()
            return carry

        lax.fori_loop(0, q // NBUF, group, 0)
        for b in range(NBUF):
            desc(q + b, b).wait()
            odesc(q - NBUF + b, b).wait()

    return body(x, nbr_flat)


BLK = 512
_PREC = lax.Precision.HIGHEST


def _tc_time_body(scal_ref, tt_ref, Tv_ref, ht_ref):
    tt = tt_ref[...]
    w0 = scal_ref[0, 0]
    b0 = scal_ref[0, 1]
    zrows = [jnp.sum(tt, axis=0, keepdims=True) * w0 + (DEG * b0)]
    for j in range(TDIM - 1):
        wt = scal_ref[0, 2 + j]
        bt = scal_ref[0, 2 + (TDIM - 1) + j]
        zrows.append(jnp.sum(jnp.sin(tt * wt + bt), axis=0, keepdims=True))
    Z = jnp.concatenate(zrows, axis=0)
    ht_ref[...] = lax.dot_general(
        Z, Tv_ref[...], (((0,), (0,)), ((), ())),
        precision=_PREC, preferred_element_type=jnp.float32)


def _tc_time(scal, times_t, Tv):
    grid = (pl.cdiv(N, BLK),)
    return pl.pallas_call(
        _tc_time_body,
        grid=grid,
        in_specs=[
            pl.BlockSpec(memory_space=pltpu.SMEM),
            pl.BlockSpec((DEG, BLK), lambda i: (0, i)),
            pl.BlockSpec((TDIM, F), lambda i: (0, 0)),
        ],
        out_specs=pl.BlockSpec((BLK, F), lambda i: (i, 0)),
        out_shape=jax.ShapeDtypeStruct((N, F), jnp.float32),
    )(scal, times_t, Tv)


def _tc_mlp_body(x_ref, g_ref, ht_ref, Wv_ref, W1a_ref, W1b_ref, W2_ref,
                 b1_ref, b2_ref, o_ref):
    h = jnp.dot(g_ref[...], Wv_ref[...], precision=_PREC,
                preferred_element_type=jnp.float32) + ht_ref[...]
    pre = (jnp.dot(x_ref[...], W1a_ref[...], precision=_PREC,
                   preferred_element_type=jnp.float32)
           + jnp.dot(h, W1b_ref[...], precision=_PREC,
                     preferred_element_type=jnp.float32)
           + b1_ref[...])
    o_ref[...] = (jnp.dot(jnp.maximum(pre, 0.0), W2_ref[...], precision=_PREC,
                          preferred_element_type=jnp.float32) + b2_ref[...])


def _tc_mlp(x, g, ht, Wv, W1a, W1b, W2, b1eff, b2r):
    grid = (pl.cdiv(N, BLK),)
    full = lambda shape: pl.BlockSpec(shape, lambda i: (0, 0))
    return pl.pallas_call(
        _tc_mlp_body,
        grid=grid,
        in_specs=[
            pl.BlockSpec((BLK, F), lambda i: (i, 0)),
            pl.BlockSpec((BLK, F), lambda i: (i, 0)),
            pl.BlockSpec((BLK, F), lambda i: (i, 0)),
            full((F, F)),
            full((F, F)),
            full((F, F)),
            full((F, F)),
            full((1, F)),
            full((1, F)),
        ],
        out_specs=pl.BlockSpec((BLK, F), lambda i: (i, 0)),
        out_shape=jax.ShapeDtypeStruct((N, F), jnp.float32),
    )(x, g, ht, Wv, W1a, W1b, W2, b1eff, b2r)


def kernel(x, neighbors, times, t, Wk, bk, Wq, bq, Wv, bv, w0, b0, Wt, Bt,
           Tk, btk, Tq, btq, Tv, btv, W1, b1, W2, b2):
    nbr = neighbors.astype(jnp.int32).reshape(-1)
    nbr_flat = jnp.pad(nbr, (0, (NPAD - N) * DEG)).reshape(NPAD // CH, ROWS)
    nbr_flat = jnp.pad(nbr_flat, ((0, Q0 - Q1), (0, 0)))
    g = _sc_gather_sum(x, nbr_flat)[:N]

    times_t = jnp.pad(times.T, ((0, 0), (0, BLK * pl.cdiv(N, BLK) - N)))
    scal = jnp.concatenate(
        [w0.reshape(1), b0.reshape(1), Wt.reshape(TDIM - 1), Bt.reshape(TDIM - 1)]
    ).reshape(1, 2 * TDIM)
    W1a = W1[:F]
    W1b = W1[F:]
    b1eff = (b1 + (DEG * (bv + btv)) @ W1b).reshape(1, F)
    ht = _tc_time(scal, times_t, Tv)
    return _tc_mlp(x, g, ht, Wv, W1a, W1b, W2, b1eff, b2.reshape(1, F))


# speedup vs baseline: 1.1164x; 1.0272x over previous
import functools

import jax
import jax.numpy as jnp
from jax import lax
from jax.experimental import pallas as pl
from jax.experimental.pallas import tpu as pltpu
from jax.experimental.pallas import tpu_sc as plsc

N = 10000
DEG = 32
F = 128
TDIM = 16

NW = 32
NODES_PER_W = 320
NPAD = NW * NODES_PER_W
CH = 8
ROWS = 128
CHUNKS = NODES_PER_W // CH
NBUF = 2


def _sc_gather_sum(x, nbr_flat):
    info = plsc.get_sparse_core_info()
    nc = info.num_cores

    mesh = plsc.VectorSubcoreMesh(core_axis_name="c", subcore_axis_name="s")

    @functools.partial(
        pl.kernel,
        mesh=mesh,
        out_type=jax.ShapeDtypeStruct((NPAD, F), jnp.float32),
        scratch_types=[
            pltpu.VMEM((2 * CHUNKS, ROWS), jnp.int32),
        ] + [pltpu.VMEM((ROWS, F), jnp.float32) for _ in range(2 * NBUF)]
          + [pltpu.VMEM((CH, F), jnp.float32) for _ in range(NBUF)]
          + [pltpu.SemaphoreType.DMA for _ in range(2 * NBUF)],
    )
    def body(x_hbm, nbr_hbm, out_hbm, idx_v, *bufs):
        rows = bufs[:2 * NBUF]
        pools = bufs[2 * NBUF:3 * NBUF]
        sems = bufs[3 * NBUF:4 * NBUF]
        osems = bufs[4 * NBUF:5 * NBUF]
        sid = lax.axis_index("s")
        wid = sid * nc + lax.axis_index("c")

        pltpu.sync_copy(nbr_hbm.at[pl.ds(wid * 2 * CHUNKS, 2 * CHUNKS)], idx_v)

        def descs(chunk_i, slot):
            ci = lax.min(chunk_i, CHUNKS - 1)
            return [
                pltpu.make_async_copy(
                    x_hbm.at[idx_v.at[2 * ci + h]], rows[2 * slot + h],
                    sems[slot])
                for h in range(2)
            ]

        def odesc(chunk_i, slot):
            base = (wid * CHUNKS + lax.min(chunk_i, CHUNKS - 1)) * CH
            return pltpu.make_async_copy(
                pools[slot], out_hbm.at[pl.ds(base, CH)], osems[slot])

        for b in range(NBUF):
            for d in descs(b, b):
                d.start()

        def group(ig, carry):
            for b in range(NBUF):
                i = ig * NBUF + b
                for d in descs(i, b):
                    d.wait()
                pb = pools[b]
                @pl.when(ig > 0)
                def _():
                    odesc(i - NBUF, b).wait()
                for h in range(2):
                    rb = rows[2 * b + h]

                    def red(gi, carry2, rb=rb, h=h):
                        c = gi // (F // 16)
                        col = gi % (F // 16)
                        sl = pl.ds(col * 16, 16)
                        base = c * DEG
                        a0 = rb[base + 0, sl]
                        a1 = rb[base + 1, sl]
                        for d in range(2, DEG, 2):
                            a0 = a0 + rb[base + d, sl]
                            a1 = a1 + rb[base + d + 1, sl]
                        pb[4 * h + c, sl] = a0 + a1
                        return carry2

                    lax.fori_loop(0, (CH // 2) * (F // 16), red, 0)
                odesc(i, b).start()
                for d in descs(i + NBUF, b):
                    d.start()
            return carry

        lax.fori_loop(0, CHUNKS // NBUF, group, 0)
        for b in range(NBUF):
            for d in descs(CHUNKS + b, b):
                d.wait()
            odesc(CHUNKS - NBUF + b, b).wait()

    return body(x, nbr_flat)


BLK = 512
_PREC = lax.Precision.HIGHEST


def _tc_time_body(scal_ref, tt_ref, Tv_ref, ht_ref):
    tt = tt_ref[...]
    w0 = scal_ref[0, 0]
    b0 = scal_ref[0, 1]
    zrows = [jnp.sum(tt, axis=0, keepdims=True) * w0 + (DEG * b0)]
    for j in range(TDIM - 1):
        wt = scal_ref[0, 2 + j]
        bt = scal_ref[0, 2 + (TDIM - 1) + j]
        zrows.append(jnp.sum(jnp.sin(tt * wt + bt), axis=0, keepdims=True))
    Z = jnp.concatenate(zrows, axis=0)
    ht_ref[...] = lax.dot_general(
        Z, Tv_ref[...], (((0,), (0,)), ((), ())),
        precision=_PREC, preferred_element_type=jnp.float32)


def _tc_time(scal, times_t, Tv):
    grid = (pl.cdiv(N, BLK),)
    return pl.pallas_call(
        _tc_time_body,
        grid=grid,
        in_specs=[
            pl.BlockSpec(memory_space=pltpu.SMEM),
            pl.BlockSpec((DEG, BLK), lambda i: (0, i)),
            pl.BlockSpec((TDIM, F), lambda i: (0, 0)),
        ],
        out_specs=pl.BlockSpec((BLK, F), lambda i: (i, 0)),
        out_shape=jax.ShapeDtypeStruct((N, F), jnp.float32),
    )(scal, times_t, Tv)


def _tc_mlp_body(x_ref, g_ref, ht_ref, Wv_ref, W1a_ref, W1b_ref, W2_ref,
                 b1_ref, b2_ref, o_ref):
    h = jnp.dot(g_ref[...], Wv_ref[...], precision=_PREC,
                preferred_element_type=jnp.float32) + ht_ref[...]
    pre = (jnp.dot(x_ref[...], W1a_ref[...], precision=_PREC,
                   preferred_element_type=jnp.float32)
           + jnp.dot(h, W1b_ref[...], precision=_PREC,
                     preferred_element_type=jnp.float32)
           + b1_ref[...])
    o_ref[...] = (jnp.dot(jnp.maximum(pre, 0.0), W2_ref[...], precision=_PREC,
                          preferred_element_type=jnp.float32) + b2_ref[...])


def _tc_mlp(x, g, ht, Wv, W1a, W1b, W2, b1eff, b2r):
    grid = (pl.cdiv(N, BLK),)
    full = lambda shape: pl.BlockSpec(shape, lambda i: (0, 0))
    return pl.pallas_call(
        _tc_mlp_body,
        grid=grid,
        in_specs=[
            pl.BlockSpec((BLK, F), lambda i: (i, 0)),
            pl.BlockSpec((BLK, F), lambda i: (i, 0)),
            pl.BlockSpec((BLK, F), lambda i: (i, 0)),
            full((F, F)),
            full((F, F)),
            full((F, F)),
            full((F, F)),
            full((1, F)),
            full((1, F)),
        ],
        out_specs=pl.BlockSpec((BLK, F), lambda i: (i, 0)),
        out_shape=jax.ShapeDtypeStruct((N, F), jnp.float32),
    )(x, g, ht, Wv, W1a, W1b, W2, b1eff, b2r)


def kernel(x, neighbors, times, t, Wk, bk, Wq, bq, Wv, bv, w0, b0, Wt, Bt,
           Tk, btk, Tq, btq, Tv, btv, W1, b1, W2, b2):
    nbr = neighbors.astype(jnp.int32).reshape(-1)
    nbr_flat = jnp.pad(nbr, (0, (NPAD - N) * DEG)).reshape(NPAD * DEG // ROWS, ROWS)
    g = _sc_gather_sum(x, nbr_flat)[:N]

    times_t = jnp.pad(times.T, ((0, 0), (0, BLK * pl.cdiv(N, BLK) - N)))
    scal = jnp.concatenate(
        [w0.reshape(1), b0.reshape(1), Wt.reshape(TDIM - 1), Bt.reshape(TDIM - 1)]
    ).reshape(1, 2 * TDIM)
    W1a = W1[:F]
    W1b = W1[F:]
    b1eff = (b1 + (DEG * (bv + btv)) @ W1b).reshape(1, F)
    ht = _tc_time(scal, times_t, Tv)
    return _tc_mlp(x, g, ht, Wv, W1a, W1b, W2, b1eff, b2.reshape(1, F))


# speedup vs baseline: 1.1251x; 1.0077x over previous
import functools

import jax
import jax.numpy as jnp
from jax import lax
from jax.experimental import pallas as pl
from jax.experimental.pallas import tpu as pltpu
from jax.experimental.pallas import tpu_sc as plsc

N = 10000
DEG = 32
F = 128
TDIM = 16

NW = 32
NODES_PER_W = 320
NPAD = NW * NODES_PER_W
CH = 8
ROWS = 128
CHUNKS = NODES_PER_W // CH
NBUF = 2
Q0 = 68
Q1 = 12


def _sc_gather_sum(x, nbr_flat):
    info = plsc.get_sparse_core_info()
    nc = info.num_cores

    mesh = plsc.VectorSubcoreMesh(core_axis_name="c", subcore_axis_name="s")

    @functools.partial(
        pl.kernel,
        mesh=mesh,
        out_type=jax.ShapeDtypeStruct((NPAD, F), jnp.float32),
        scratch_types=[
            pltpu.VMEM((2 * Q0, ROWS), jnp.int32),
        ] + [pltpu.VMEM((ROWS, F), jnp.float32) for _ in range(2 * NBUF)]
          + [pltpu.VMEM((CH, F), jnp.float32) for _ in range(NBUF)]
          + [pltpu.SemaphoreType.DMA for _ in range(2 * NBUF)],
    )
    def body(x_hbm, nbr_hbm, out_hbm, idx_v, *bufs):
        rows = bufs[:2 * NBUF]
        pools = bufs[2 * NBUF:3 * NBUF]
        sems = bufs[3 * NBUF:4 * NBUF]
        osems = bufs[4 * NBUF:5 * NBUF]
        sid = lax.axis_index("s")
        cid = lax.axis_index("c")
        q = lax.select(cid == 0, jnp.int32(Q0), jnp.int32(Q1))
        start_chunk = lax.select(cid == 0, sid * Q0, 16 * Q0 + sid * Q1)

        pltpu.sync_copy(nbr_hbm.at[pl.ds(start_chunk * 2, 2 * Q0)], idx_v)

        def descs(chunk_i, slot):
            ci = lax.min(chunk_i, q - 1)
            return [
                pltpu.make_async_copy(
                    x_hbm.at[idx_v.at[2 * ci + h]], rows[2 * slot + h],
                    sems[slot])
                for h in range(2)
            ]

        def odesc(chunk_i, slot):
            base = (start_chunk + lax.min(chunk_i, q - 1)) * CH
            return pltpu.make_async_copy(
                pools[slot], out_hbm.at[pl.ds(base, CH)], osems[slot])

        for b in range(NBUF):
            for d in descs(b, b):
                d.start()

        def group(ig, carry):
            for b in range(NBUF):
                i = ig * NBUF + b
                for d in descs(i, b):
                    d.wait()
                pb = pools[b]
                @pl.when(ig > 0)
                def _():
                    odesc(i - NBUF, b).wait()
                for h in range(2):
                    rb = rows[2 * b + h]

                    def red(gi, carry2, rb=rb, h=h):
                        c = gi // (F // 16)
                        col = gi % (F // 16)
                        sl = pl.ds(col * 16, 16)
                        base = c * DEG
                        a0 = rb[base + 0, sl]
                        a1 = rb[base + 1, sl]
                        for d in range(2, DEG, 2):
                            a0 = a0 + rb[base + d, sl]
                            a1 = a1 + rb[base + d + 1, sl]
                        pb[4 * h + c, sl] = a0 + a1
                        return carry2

                    lax.fori_loop(0, (CH // 2) * (F // 16), red, 0)
                odesc(i, b).start()
                for d in descs(i + NBUF, b):
                    d.start()
            return carry

        lax.fori_loop(0, q // NBUF, group, 0)
        for b in range(NBUF):
            for d in descs(q + b, b):
                d.wait()
            odesc(q - NBUF + b, b).wait()

    return body(x, nbr_flat)


BLK = 512
_PREC = lax.Precision.HIGHEST


def _tc_time_body(scal_ref, tt_ref, Tv_ref, ht_ref):
    tt = tt_ref[...]
    w0 = scal_ref[0, 0]
    b0 = scal_ref[0, 1]
    zrows = [jnp.sum(tt, axis=0, keepdims=True) * w0 + (DEG * b0)]
    for j in range(TDIM - 1):
        wt = scal_ref[0, 2 + j]
        bt = scal_ref[0, 2 + (TDIM - 1) + j]
        zrows.append(jnp.sum(jnp.sin(tt * wt + bt), axis=0, keepdims=True))
    Z = jnp.concatenate(zrows, axis=0)
    ht_ref[...] = lax.dot_general(
        Z, Tv_ref[...], (((0,), (0,)), ((), ())),
        precision=_PREC, preferred_element_type=jnp.float32)


def _tc_time(scal, times_t, Tv):
    grid = (pl.cdiv(N, BLK),)
    return pl.pallas_call(
        _tc_time_body,
        grid=grid,
        in_specs=[
            pl.BlockSpec(memory_space=pltpu.SMEM),
            pl.BlockSpec((DEG, BLK), lambda i: (0, i)),
            pl.BlockSpec((TDIM, F), lambda i: (0, 0)),
        ],
        out_specs=pl.BlockSpec((BLK, F), lambda i: (i, 0)),
        out_shape=jax.ShapeDtypeStruct((N, F), jnp.float32),
    )(scal, times_t, Tv)


def _tc_mlp_body(x_ref, g_ref, ht_ref, Wv_ref, W1a_ref, W1b_ref, W2_ref,
                 b1_ref, b2_ref, o_ref):
    h = jnp.dot(g_ref[...], Wv_ref[...], precision=_PREC,
                preferred_element_type=jnp.float32) + ht_ref[...]
    pre = (jnp.dot(x_ref[...], W1a_ref[...], precision=_PREC,
                   preferred_element_type=jnp.float32)
           + jnp.dot(h, W1b_ref[...], precision=_PREC,
                     preferred_element_type=jnp.float32)
           + b1_ref[...])
    o_ref[...] = (jnp.dot(jnp.maximum(pre, 0.0), W2_ref[...], precision=_PREC,
                          preferred_element_type=jnp.float32) + b2_ref[...])


def _tc_mlp(x, g, ht, Wv, W1a, W1b, W2, b1eff, b2r):
    grid = (pl.cdiv(N, BLK),)
    full = lambda shape: pl.BlockSpec(shape, lambda i: (0, 0))
    return pl.pallas_call(
        _tc_mlp_body,
        grid=grid,
        in_specs=[
            pl.BlockSpec((BLK, F), lambda i: (i, 0)),
            pl.BlockSpec((BLK, F), lambda i: (i, 0)),
            pl.BlockSpec((BLK, F), lambda i: (i, 0)),
            full((F, F)),
            full((F, F)),
            full((F, F)),
            full((F, F)),
            full((1, F)),
            full((1, F)),
        ],
        out_specs=pl.BlockSpec((BLK, F), lambda i: (i, 0)),
        out_shape=jax.ShapeDtypeStruct((N, F), jnp.float32),
    )(x, g, ht, Wv, W1a, W1b, W2, b1eff, b2r)


def kernel(x, neighbors, times, t, Wk, bk, Wq, bq, Wv, bv, w0, b0, Wt, Bt,
           Tk, btk, Tq, btq, Tv, btv, W1, b1, W2, b2):
    nbr = neighbors.astype(jnp.int32).reshape(-1)
    nbr_flat = jnp.pad(nbr, (0, (NPAD - N) * DEG)).reshape(NPAD * DEG // ROWS, ROWS)
    nbr_flat = jnp.pad(nbr_flat, ((0, 2 * Q0 - 2 * Q1), (0, 0)))
    g = _sc_gather_sum(x, nbr_flat)

    times_t = jnp.pad(times.T, ((0, 0), (0, BLK * pl.cdiv(N, BLK) - N)))
    scal = jnp.concatenate(
        [w0.reshape(1), b0.reshape(1), Wt.reshape(TDIM - 1), Bt.reshape(TDIM - 1)]
    ).reshape(1, 2 * TDIM)
    W1a = W1[:F]
    W1b = W1[F:]
    b1eff = (b1 + (DEG * (bv + btv)) @ W1b).reshape(1, F)
    ht = _tc_time(scal, times_t, Tv)
    return _tc_mlp(x, g, ht, Wv, W1a, W1b, W2, b1eff, b2.reshape(1, F))
